# baseline (speedup 1.0000x reference)
import functools

import jax
import jax.numpy as jnp
from jax.experimental import pallas as pl
from jax.experimental.pallas import tpu as pltpu

_NEG_SLOPE = 0.1
_BN_EPS = 1e-5
_VMEM_LIMIT = 48 * 1024 * 1024


def _leaky(v):
    return jnp.where(v >= 0, v, _NEG_SLOPE * v)


def _csp_kernel(x_ref, wh_ref, bh_ref, w1s_ref, b1s_ref, w2s_ref, b2s_ref,
                wza_ref, wzb_ref, bz_ref, w4_ref, b4_ref,
                o_ref, c2_ref, *, H, W, n_blocks):
    HW = H * W
    c_ = wh_ref.shape[1]
    xc = x_ref[0].astype(jnp.bfloat16)

    y = _leaky(jax.lax.dot_general(
        xc, wh_ref[...], (((0,), (0,)), ((), ())),
        preferred_element_type=jnp.float32) + bh_ref[...])

    pad = W + 8
    col = jax.lax.broadcasted_iota(jnp.int32, (HW, 1), 0) % W
    not_left = col != 0
    not_right = col != (W - 1)

    for blk in range(n_blocks):
        yb = y.astype(jnp.bfloat16)
        t = _leaky(jnp.dot(yb, w1s_ref[blk], preferred_element_type=jnp.float32)
                   + b1s_ref[blk])
        tb = t.astype(jnp.bfloat16)
        zeros = jnp.zeros((pad, c_), jnp.bfloat16)
        tpad = jnp.concatenate([zeros, tb, zeros], axis=0)

        for kh in range(3):
            for kw in range(3):
                dh, dw = kh - 1, kw - 1
                start = pad + dh * W + dw
                win = tpad[start:start + HW, :]
                if dw == -1:
                    win = jnp.where(not_left, win, jnp.bfloat16(0))
                elif dw == 1:
                    win = jnp.where(not_right, win, jnp.bfloat16(0))
                tap = kh * 3 + kw
                c2_ref[:, tap * c_:(tap + 1) * c_] = win
        acc = jnp.dot(c2_ref[...], w2s_ref[blk],
                      preferred_element_type=jnp.float32)
        y = _leaky(acc + b2s_ref[blk]) + y

    yb = y.astype(jnp.bfloat16)
    u = _leaky(jnp.dot(yb, wza_ref[...], preferred_element_type=jnp.float32)
               + jax.lax.dot_general(
                   xc, wzb_ref[...], (((0,), (0,)), ((), ())),
                   preferred_element_type=jnp.float32)
               + bz_ref[...]).astype(jnp.bfloat16)
    vt = jax.lax.dot_general(
        w4_ref[...], u, (((0,), (1,)), ((), ())),
        preferred_element_type=jnp.float32)
    o_ref[0] = _leaky(vt + b4_ref[...])


def _w1x1(w):
    return jnp.transpose(w[:, :, 0, 0], (1, 0))


def _fold_scale(gamma, var):
    return gamma * jax.lax.rsqrt(var + _BN_EPS)


def kernel(x, cv1_conv_w, cv1_conv_b, cv1_bn_gamma, cv1_bn_beta, cv1_bn_mean, cv1_bn_var, cv2_w, cv3_w, cv4_conv_w, cv4_conv_b, cv4_bn_gamma, cv4_bn_beta, cv4_bn_mean, cv4_bn_var, bn_gamma, bn_beta, bn_mean, bn_var, m0_cv1_conv_w, m0_cv1_conv_b, m0_cv1_bn_gamma, m0_cv1_bn_beta, m0_cv1_bn_mean, m0_cv1_bn_var, m0_cv2_conv_w, m0_cv2_conv_b, m0_cv2_bn_gamma, m0_cv2_bn_beta, m0_cv2_bn_mean, m0_cv2_bn_var, m1_cv1_conv_w, m1_cv1_conv_b, m1_cv1_bn_gamma, m1_cv1_bn_beta, m1_cv1_bn_mean, m1_cv1_bn_var, m1_cv2_conv_w, m1_cv2_conv_b, m1_cv2_bn_gamma, m1_cv2_bn_beta, m1_cv2_bn_mean, m1_cv2_bn_var, m2_cv1_conv_w, m2_cv1_conv_b, m2_cv1_bn_gamma, m2_cv1_bn_beta, m2_cv1_bn_mean, m2_cv1_bn_var, m2_cv2_conv_w, m2_cv2_conv_b, m2_cv2_bn_gamma, m2_cv2_bn_beta, m2_cv2_bn_mean, m2_cv2_bn_var):
    Nb, c1, H, W = x.shape
    HW = H * W
    M = Nb * HW

    s_h = _fold_scale(cv1_bn_gamma, cv1_bn_var)
    wh = (_w1x1(cv1_conv_w) * s_h[None, :]).astype(jnp.bfloat16)
    bh = (s_h * (cv1_conv_b - cv1_bn_mean) + cv1_bn_beta).reshape(1, -1)
    c_ = wh.shape[1]

    blocks = [
        (m0_cv1_conv_w, m0_cv1_conv_b, m0_cv1_bn_gamma, m0_cv1_bn_beta,
         m0_cv1_bn_mean, m0_cv1_bn_var, m0_cv2_conv_w, m0_cv2_conv_b,
         m0_cv2_bn_gamma, m0_cv2_bn_beta, m0_cv2_bn_mean, m0_cv2_bn_var),
        (m1_cv1_conv_w, m1_cv1_conv_b, m1_cv1_bn_gamma, m1_cv1_bn_beta,
         m1_cv1_bn_mean, m1_cv1_bn_var, m1_cv2_conv_w, m1_cv2_conv_b,
         m1_cv2_bn_gamma, m1_cv2_bn_beta, m1_cv2_bn_mean, m1_cv2_bn_var),
        (m2_cv1_conv_w, m2_cv1_conv_b, m2_cv1_bn_gamma, m2_cv1_bn_beta,
         m2_cv1_bn_mean, m2_cv1_bn_var, m2_cv2_conv_w, m2_cv2_conv_b,
         m2_cv2_bn_gamma, m2_cv2_bn_beta, m2_cv2_bn_mean, m2_cv2_bn_var),
    ]
    w1s, b1s, w2s, b2s = [], [], [], []
    for (w1, b1, g1, be1, mu1, v1, w2, b2, g2, be2, mu2, v2) in blocks:
        s1 = _fold_scale(g1, v1)
        w1s.append((_w1x1(w1) * s1[None, :]).astype(jnp.bfloat16))
        b1s.append((s1 * (b1 - mu1) + be1).reshape(1, -1))
        s2 = _fold_scale(g2, v2)
        taps = jnp.transpose(w2, (2, 3, 1, 0)).reshape(9, w2.shape[1], w2.shape[0])
        w2s.append((taps * s2[None, None, :]).astype(jnp.bfloat16))
        b2s.append((s2 * (b2 - mu2) + be2).reshape(1, -1))
    n_blocks = len(blocks)
    w1s = jnp.stack(w1s)
    b1s = jnp.stack(b1s)
    w2s = jnp.stack([w.reshape(9 * w.shape[1], w.shape[2]) for w in w2s])
    b2s = jnp.stack(b2s)

    s_bn = _fold_scale(bn_gamma, bn_var)
    b_bn = bn_beta - bn_mean * s_bn
    w3 = _w1x1(cv3_w) * s_bn[None, :c_]
    w2o = _w1x1(cv2_w) * s_bn[None, c_:]
    wza = jnp.concatenate(
        [w3, jnp.zeros((c_, c_), jnp.float32)], axis=1).astype(jnp.bfloat16)
    wzb = jnp.concatenate(
        [jnp.zeros((c1, c_), jnp.float32), w2o], axis=1).astype(jnp.bfloat16)
    bz = b_bn.reshape(1, -1)
    s4 = _fold_scale(cv4_bn_gamma, cv4_bn_var)
    w4 = (_w1x1(cv4_conv_w) * s4[None, :]).astype(jnp.bfloat16)
    b4 = (s4 * (cv4_conv_b - cv4_bn_mean) + cv4_bn_beta).reshape(-1, 1)
    c2 = w4.shape[1]

    x3d = x.reshape(Nb, c1, HW)

    kern = functools.partial(_csp_kernel, H=H, W=W, n_blocks=n_blocks)
    rep = lambda i: (0, 0)
    rep3 = lambda i: (0, 0, 0)
    out = pl.pallas_call(
        kern,
        out_shape=jax.ShapeDtypeStruct((Nb, c2, HW), jnp.float32),
        grid_spec=pltpu.PrefetchScalarGridSpec(
            num_scalar_prefetch=0,
            grid=(Nb,),
            in_specs=[
                pl.BlockSpec((1, c1, HW), lambda i: (i, 0, 0)),
                pl.BlockSpec(wh.shape, rep), pl.BlockSpec(bh.shape, rep),
                pl.BlockSpec(w1s.shape, rep3), pl.BlockSpec(b1s.shape, rep3),
                pl.BlockSpec(w2s.shape, rep3), pl.BlockSpec(b2s.shape, rep3),
                pl.BlockSpec(wza.shape, rep), pl.BlockSpec(wzb.shape, rep),
                pl.BlockSpec(bz.shape, rep),
                pl.BlockSpec(w4.shape, rep), pl.BlockSpec(b4.shape, rep),
            ],
            out_specs=pl.BlockSpec((1, c2, HW), lambda i: (i, 0, 0)),
            scratch_shapes=[pltpu.VMEM((HW, 9 * c_), jnp.bfloat16)],
        ),
        compiler_params=pltpu.CompilerParams(
            dimension_semantics=("parallel",), vmem_limit_bytes=_VMEM_LIMIT),
    )(x3d, wh, bh, w1s, b1s, w2s, b2s, wza, wzb, bz, w4, b4)

    return out.reshape(Nb, c2, H, W)


# speedup vs baseline: 1.6927x; 1.6927x over previous
import functools

import jax
import jax.numpy as jnp
from jax.experimental import pallas as pl
from jax.experimental.pallas import tpu as pltpu

_NEG_SLOPE = 0.1
_BN_EPS = 1e-5
_VMEM_LIMIT = 48 * 1024 * 1024


def _leaky(v):
    return jnp.where(v >= 0, v, _NEG_SLOPE * v)


def _csp_kernel(x_ref, wh_ref, bh_ref, w1s_ref, b1s_ref, w2s_ref, b2s_ref,
                wz_ref, bz_ref, w4_ref, b4_ref,
                o_ref, c3_ref, *, H, W, n_blocks):
    HW = H * W
    c_ = wh_ref.shape[1]
    xb = x_ref[...].astype(jnp.bfloat16)

    y = _leaky(jnp.dot(xb, wh_ref[...], preferred_element_type=jnp.float32)
               + bh_ref[...])

    col = jax.lax.broadcasted_iota(jnp.int32, (HW, 1), 0) % W
    not_left = col != 0
    not_right = col != (W - 1)

    c3_ref[0:W + 1, :] = jnp.zeros((W + 1, 3 * c_), jnp.bfloat16)
    c3_ref[W + HW - 1:, :] = jnp.zeros((W + 1, 3 * c_), jnp.bfloat16)

    for blk in range(n_blocks):
        yb = y.astype(jnp.bfloat16)
        tb = _leaky((jnp.dot(yb, w1s_ref[blk], preferred_element_type=jnp.float32)
                     + b1s_ref[blk]).astype(jnp.bfloat16))
        tl = jnp.where(not_right, tb, jnp.bfloat16(0))
        tr = jnp.where(not_left, tb, jnp.bfloat16(0))
        c3_ref[W + 1:W + 1 + HW, 0:c_] = tl
        c3_ref[W:W + HW, c_:2 * c_] = tb
        c3_ref[W - 1:W - 1 + HW, 2 * c_:3 * c_] = tr
        acc = jnp.dot(c3_ref[0:HW, :], w2s_ref[3 * blk],
                      preferred_element_type=jnp.float32)
        acc = acc + jnp.dot(c3_ref[W:W + HW, :], w2s_ref[3 * blk + 1],
                            preferred_element_type=jnp.float32)
        acc = acc + jnp.dot(c3_ref[2 * W:2 * W + HW, :], w2s_ref[3 * blk + 2],
                            preferred_element_type=jnp.float32)
        y = _leaky(acc + b2s_ref[blk]) + y

    yb = y.astype(jnp.bfloat16)
    zin = jnp.concatenate([yb, xb], axis=1)
    u = _leaky((jnp.dot(zin, wz_ref[...], preferred_element_type=jnp.float32)
                + bz_ref[...]).astype(jnp.bfloat16))
    v = jnp.dot(u, w4_ref[...], preferred_element_type=jnp.float32) + b4_ref[...]
    o_ref[...] = _leaky(v)


def _w1x1(w):
    return jnp.transpose(w[:, :, 0, 0], (1, 0))


def _fold_scale(gamma, var):
    return gamma * jax.lax.rsqrt(var + _BN_EPS)


def kernel(x, cv1_conv_w, cv1_conv_b, cv1_bn_gamma, cv1_bn_beta, cv1_bn_mean, cv1_bn_var, cv2_w, cv3_w, cv4_conv_w, cv4_conv_b, cv4_bn_gamma, cv4_bn_beta, cv4_bn_mean, cv4_bn_var, bn_gamma, bn_beta, bn_mean, bn_var, m0_cv1_conv_w, m0_cv1_conv_b, m0_cv1_bn_gamma, m0_cv1_bn_beta, m0_cv1_bn_mean, m0_cv1_bn_var, m0_cv2_conv_w, m0_cv2_conv_b, m0_cv2_bn_gamma, m0_cv2_bn_beta, m0_cv2_bn_mean, m0_cv2_bn_var, m1_cv1_conv_w, m1_cv1_conv_b, m1_cv1_bn_gamma, m1_cv1_bn_beta, m1_cv1_bn_mean, m1_cv1_bn_var, m1_cv2_conv_w, m1_cv2_conv_b, m1_cv2_bn_gamma, m1_cv2_bn_beta, m1_cv2_bn_mean, m1_cv2_bn_var, m2_cv1_conv_w, m2_cv1_conv_b, m2_cv1_bn_gamma, m2_cv1_bn_beta, m2_cv1_bn_mean, m2_cv1_bn_var, m2_cv2_conv_w, m2_cv2_conv_b, m2_cv2_bn_gamma, m2_cv2_bn_beta, m2_cv2_bn_mean, m2_cv2_bn_var):
    Nb, c1, H, W = x.shape
    HW = H * W
    M = Nb * HW

    s_h = _fold_scale(cv1_bn_gamma, cv1_bn_var)
    wh = (_w1x1(cv1_conv_w) * s_h[None, :]).astype(jnp.bfloat16)
    bh = (s_h * (cv1_conv_b - cv1_bn_mean) + cv1_bn_beta).reshape(1, -1)
    c_ = wh.shape[1]

    blocks = [
        (m0_cv1_conv_w, m0_cv1_conv_b, m0_cv1_bn_gamma, m0_cv1_bn_beta,
         m0_cv1_bn_mean, m0_cv1_bn_var, m0_cv2_conv_w, m0_cv2_conv_b,
         m0_cv2_bn_gamma, m0_cv2_bn_beta, m0_cv2_bn_mean, m0_cv2_bn_var),
        (m1_cv1_conv_w, m1_cv1_conv_b, m1_cv1_bn_gamma, m1_cv1_bn_beta,
         m1_cv1_bn_mean, m1_cv1_bn_var, m1_cv2_conv_w, m1_cv2_conv_b,
         m1_cv2_bn_gamma, m1_cv2_bn_beta, m1_cv2_bn_mean, m1_cv2_bn_var),
        (m2_cv1_conv_w, m2_cv1_conv_b, m2_cv1_bn_gamma, m2_cv1_bn_beta,
         m2_cv1_bn_mean, m2_cv1_bn_var, m2_cv2_conv_w, m2_cv2_conv_b,
         m2_cv2_bn_gamma, m2_cv2_bn_beta, m2_cv2_bn_mean, m2_cv2_bn_var),
    ]
    w1s, b1s, w2s, b2s = [], [], [], []
    for (w1, b1, g1, be1, mu1, v1, w2, b2, g2, be2, mu2, v2) in blocks:
        s1 = _fold_scale(g1, v1)
        w1s.append((_w1x1(w1) * s1[None, :]).astype(jnp.bfloat16))
        b1s.append((s1 * (b1 - mu1) + be1).reshape(1, -1))
        s2 = _fold_scale(g2, v2)
        taps = jnp.transpose(w2, (2, 3, 1, 0)).reshape(9, w2.shape[1], w2.shape[0])
        w2s.append((taps * s2[None, None, :]).astype(jnp.bfloat16))
        b2s.append((s2 * (b2 - mu2) + be2).reshape(1, -1))
    n_blocks = len(blocks)
    w1s = jnp.stack(w1s)
    b1s = jnp.stack(b1s)
    w2s = jnp.concatenate(
        [w.reshape(3, 3 * w.shape[1], w.shape[2]) for w in w2s], axis=0)
    b2s = jnp.stack(b2s)

    s_bn = _fold_scale(bn_gamma, bn_var)
    b_bn = bn_beta - bn_mean * s_bn
    w3 = _w1x1(cv3_w) * s_bn[None, :c_]
    w2o = _w1x1(cv2_w) * s_bn[None, c_:]
    wz = jnp.zeros((c_ + w2o.shape[0], 2 * c_), jnp.float32)
    wz = wz.at[:c_, :c_].set(w3).at[c_:, c_:].set(w2o).astype(jnp.bfloat16)
    bz = b_bn.reshape(1, -1)
    s4 = _fold_scale(cv4_bn_gamma, cv4_bn_var)
    w4 = (_w1x1(cv4_conv_w) * s4[None, :]).astype(jnp.bfloat16)
    b4 = (s4 * (cv4_conv_b - cv4_bn_mean) + cv4_bn_beta).reshape(1, -1)
    c2 = w4.shape[1]

    x2d = jnp.transpose(x, (0, 2, 3, 1)).reshape(M, c1)

    kern = functools.partial(_csp_kernel, H=H, W=W, n_blocks=n_blocks)
    rep = lambda i: (0, 0)
    rep3 = lambda i: (0, 0, 0)
    out = pl.pallas_call(
        kern,
        out_shape=jax.ShapeDtypeStruct((M, c2), jnp.float32),
        grid_spec=pltpu.PrefetchScalarGridSpec(
            num_scalar_prefetch=0,
            grid=(Nb,),
            in_specs=[
                pl.BlockSpec((HW, c1), lambda i: (i, 0)),
                pl.BlockSpec(wh.shape, rep), pl.BlockSpec(bh.shape, rep),
                pl.BlockSpec(w1s.shape, rep3), pl.BlockSpec(b1s.shape, rep3),
                pl.BlockSpec(w2s.shape, rep3), pl.BlockSpec(b2s.shape, rep3),
                pl.BlockSpec(wz.shape, rep), pl.BlockSpec(bz.shape, rep),
                pl.BlockSpec(w4.shape, rep), pl.BlockSpec(b4.shape, rep),
            ],
            out_specs=pl.BlockSpec((HW, c2), lambda i: (i, 0)),
            scratch_shapes=[pltpu.VMEM((HW + 2 * W, 3 * c_), jnp.bfloat16)],
        ),
        compiler_params=pltpu.CompilerParams(
            dimension_semantics=("parallel",), vmem_limit_bytes=_VMEM_LIMIT),
    )(x2d, wh, bh, w1s, b1s, w2s, b2s, wz, bz, w4, b4)

    return jnp.transpose(out.reshape(Nb, H, W, c2), (0, 3, 1, 2))
